# baseline (speedup 1.0000x reference)
import functools

import jax
import jax.numpy as jnp
from jax import lax
from jax.experimental import pallas as pl
from jax.experimental.pallas import tpu as pltpu
from jax.experimental.pallas import tpu_sc as plsc

B, S = 4096, 200
D_MODEL = 128
N = B * S
NC, NS = 2, 16
NW = NC * NS
PER_W = N // NW
CH = 128
SUP = 40
N_SUP = PER_W // (CH * SUP)
NSLOT = 6
GAHEAD = 2
R_TAB = 7 * 12 * 4 * 24 * 4
R_BLK = R_TAB // 8
IDX_R = N // 128


def _prep_body(d, m, q, h, s, dow_e, mon_e, qtr_e, hr_e, ses_e, w, b2,
               table_out, fidx_out):
    g = pl.program_id(0)
    fidx_out[...] = (((d[...] * 12 + m[...]) * 4 + q[...]) * 24
                     + h[...]) * 4 + s[...]

    r = lax.broadcasted_iota(jnp.int32, (R_BLK, 1), 0) + g * R_BLK
    d_id = r // 4608
    m_id = (r // 384) % 12
    q_id = (r // 96) % 4
    h_id = (r // 4) % 24
    s_id = r % 4

    def onehot(ids, n):
        cols = lax.broadcasted_iota(jnp.int32, (R_BLK, n), 1)
        return (ids == cols).astype(jnp.float32)

    p_d = jnp.dot(dow_e[...], w[0:16, :], preferred_element_type=jnp.float32)
    p_m = jnp.dot(mon_e[...], w[16:32, :], preferred_element_type=jnp.float32)
    p_q = jnp.dot(qtr_e[...], w[32:40, :], preferred_element_type=jnp.float32)
    p_h = jnp.dot(hr_e[...], w[40:56, :], preferred_element_type=jnp.float32)
    p_s = jnp.dot(ses_e[...], w[56:64, :], preferred_element_type=jnp.float32)

    acc = jnp.dot(onehot(d_id, 7), p_d, preferred_element_type=jnp.float32)
    acc += jnp.dot(onehot(m_id, 12), p_m, preferred_element_type=jnp.float32)
    acc += jnp.dot(onehot(q_id, 4), p_q, preferred_element_type=jnp.float32)
    acc += jnp.dot(onehot(h_id, 24), p_h, preferred_element_type=jnp.float32)
    acc += jnp.dot(onehot(s_id, 4), p_s, preferred_element_type=jnp.float32)
    table_out[...] = acc + b2[...]


def _prep(d, m, q, h, s, dow_e, mon_e, qtr_e, hr_e, ses_e, w, b2):
    ib = pl.BlockSpec((IDX_R // 8, 128), lambda i: (i, 0))
    full = lambda shape: pl.BlockSpec(shape, lambda i: (0, 0))
    return pl.pallas_call(
        _prep_body,
        grid=(8,),
        in_specs=[ib] * 5 + [full((7, 16)), full((12, 16)), full((4, 8)),
                             full((24, 16)), full((4, 8)), full((64, 128)),
                             full((1, 128))],
        out_specs=[pl.BlockSpec((R_BLK, D_MODEL), lambda i: (i, 0)), ib],
        out_shape=[jax.ShapeDtypeStruct((R_TAB, D_MODEL), jnp.float32),
                   jax.ShapeDtypeStruct((IDX_R, 128), jnp.int32)],
    )(d, m, q, h, s, dow_e, mon_e, qtr_e, hr_e, ses_e, w, b2)


_mesh = plsc.VectorSubcoreMesh(core_axis_name="c", subcore_axis_name="s")


@functools.partial(
    pl.kernel,
    mesh=_mesh,
    out_type=jax.ShapeDtypeStruct((N, D_MODEL), jnp.float32),
    scratch_types=[
        pltpu.VMEM((SUP, 128), jnp.int32),
        [pltpu.VMEM((CH, D_MODEL), jnp.float32)] * NSLOT,
        [pltpu.SemaphoreType.DMA] * NSLOT,
        [pltpu.SemaphoreType.DMA] * NSLOT,
    ],
)
def _sc_gather(fidx, table, out, idx_v, rows, semg, sems):
    wid = lax.axis_index("s") * NC + lax.axis_index("c")
    base = wid * PER_W
    base_i = wid * (PER_W // 128)

    def super_body(sup, carry):
        pltpu.sync_copy(fidx.at[pl.ds(base_i + sup * SUP, SUP)], idx_v)
        off0 = base + sup * SUP * CH
        g_h = {}
        s_h = {}

        def scatter(p):
            g_h[p].wait()
            s_h[p] = pltpu.async_copy(
                rows[p % NSLOT], out.at[pl.ds(off0 + p * CH, CH)],
                sems[p % NSLOT])

        for j in range(SUP):
            slot = j % NSLOT
            if j >= NSLOT:
                s_h[j - NSLOT].wait()
            g_h[j] = pltpu.async_copy(table.at[idx_v.at[j]], rows[slot],
                                      semg[slot])
            if j >= GAHEAD:
                scatter(j - GAHEAD)
        for p in range(SUP - GAHEAD, SUP):
            scatter(p)
        for p in range(SUP - NSLOT, SUP):
            s_h[p].wait()
        return carry

    lax.fori_loop(0, N_SUP, super_body, 0)


def kernel(dayofweek, month, quarter, hour, session, dow_emb, month_emb,
           quarter_emb, hour_emb, session_emb, W, b):
    as_idx = lambda a: a.reshape(IDX_R, 128).astype(jnp.int32)
    table, fidx = _prep(as_idx(dayofweek), as_idx(month), as_idx(quarter),
                        as_idx(hour), as_idx(session), dow_emb, month_emb,
                        quarter_emb, hour_emb, session_emb, W,
                        b.reshape(1, D_MODEL))
    out = _sc_gather(fidx, table)
    return out.reshape(B, S, D_MODEL)


# speedup vs baseline: 1.1623x; 1.1623x over previous
import functools

import jax
import jax.numpy as jnp
from jax import lax
from jax.experimental import pallas as pl
from jax.experimental.pallas import tpu as pltpu
from jax.experimental.pallas import tpu_sc as plsc

B, S = 4096, 200
D_MODEL = 128
N = B * S
NC, NS = 2, 16
NW = NC * NS
BR_W = B // NW
SUPB = 16
N_SUP = BR_W // SUPB
NSLOT = 6
GAHEAD = 2
R_TAB = 7 * 12 * 4 * 24 * 4
R_BLK = R_TAB // 7


def _table_body(dow_e, mon_e, qtr_e, hr_e, ses_e, w, b2, out):
    d = pl.program_id(0)
    r = lax.broadcasted_iota(jnp.int32, (R_BLK, 1), 0)
    d_id = jnp.full((R_BLK, 1), d, jnp.int32)
    m_id = r // 384
    q_id = (r // 96) % 4
    h_id = (r // 4) % 24
    s_id = r % 4

    def onehot(ids, n):
        cols = lax.broadcasted_iota(jnp.int32, (R_BLK, n), 1)
        return (ids == cols).astype(jnp.float32)

    p_d = jnp.dot(dow_e[...], w[0:16, :], preferred_element_type=jnp.float32)
    p_m = jnp.dot(mon_e[...], w[16:32, :], preferred_element_type=jnp.float32)
    p_q = jnp.dot(qtr_e[...], w[32:40, :], preferred_element_type=jnp.float32)
    p_h = jnp.dot(hr_e[...], w[40:56, :], preferred_element_type=jnp.float32)
    p_s = jnp.dot(ses_e[...], w[56:64, :], preferred_element_type=jnp.float32)

    acc = jnp.dot(onehot(d_id, 7), p_d, preferred_element_type=jnp.float32)
    acc += jnp.dot(onehot(m_id, 12), p_m, preferred_element_type=jnp.float32)
    acc += jnp.dot(onehot(q_id, 4), p_q, preferred_element_type=jnp.float32)
    acc += jnp.dot(onehot(h_id, 24), p_h, preferred_element_type=jnp.float32)
    acc += jnp.dot(onehot(s_id, 4), p_s, preferred_element_type=jnp.float32)
    out[...] = acc + b2[...]


def _build_table(dow_e, mon_e, qtr_e, hr_e, ses_e, w, b2):
    full = lambda shape: pl.BlockSpec(shape, lambda d: (0, 0))
    return pl.pallas_call(
        _table_body,
        grid=(7,),
        in_specs=[full((7, 16)), full((12, 16)), full((4, 8)),
                  full((24, 16)), full((4, 8)), full((64, 128)),
                  full((1, 128))],
        out_specs=pl.BlockSpec((R_BLK, D_MODEL), lambda d: (d, 0)),
        out_shape=jax.ShapeDtypeStruct((R_TAB, D_MODEL), jnp.float32),
    )(dow_e, mon_e, qtr_e, hr_e, ses_e, w, b2)


def _fuse_body(d, m, q, h, s, out):
    out[...] = (((d[...] * 12 + m[...]) * 4 + q[...]) * 24 + h[...]) * 4 + s[...]


def _fuse_idx(d, m, q, h, s):
    blk = pl.BlockSpec((B // 8, S), lambda i: (i, 0))
    return pl.pallas_call(
        _fuse_body,
        grid=(8,),
        in_specs=[blk] * 5,
        out_specs=blk,
        out_shape=jax.ShapeDtypeStruct((B, S), jnp.int32),
    )(d, m, q, h, s)


_mesh = plsc.VectorSubcoreMesh(core_axis_name="c", subcore_axis_name="s")


@functools.partial(
    pl.kernel,
    mesh=_mesh,
    out_type=jax.ShapeDtypeStruct((N, D_MODEL), jnp.float32),
    scratch_types=[
        pltpu.VMEM((SUPB, S), jnp.int32),
        [pltpu.VMEM((128, D_MODEL), jnp.float32)] * NSLOT,
        [pltpu.SemaphoreType.DMA] * NSLOT,
        [pltpu.SemaphoreType.DMA] * NSLOT,
    ],
)
def _sc_gather(fidx, table, out, idx_v, rows, semg, sems):
    wid = lax.axis_index("s") * NC + lax.axis_index("c")
    base_br = wid * BR_W
    obase = wid * BR_W * S

    chunks = []
    for j in range(SUPB):
        chunks.append((j, 0, 128))
        chunks.append((j, 128, 72))

    def super_body(sup, carry):
        pltpu.sync_copy(fidx.at[pl.ds(base_br + sup * SUPB, SUPB)], idx_v)
        off0 = obase + sup * SUPB * S
        g_h = {}
        s_h = {}

        def scatter(p):
            j, ioff, size = chunks[p]
            g_h[p].wait()
            s_h[p] = pltpu.async_copy(
                rows[p % NSLOT].at[pl.ds(0, size)],
                out.at[pl.ds(off0 + j * S + ioff, size)],
                sems[p % NSLOT])

        for k in range(len(chunks)):
            j, ioff, size = chunks[k]
            slot = k % NSLOT
            if k >= NSLOT:
                s_h[k - NSLOT].wait()
            g_h[k] = pltpu.async_copy(
                table.at[idx_v.at[j, pl.ds(ioff, size)]],
                rows[slot].at[pl.ds(0, size)],
                semg[slot])
            if k >= GAHEAD:
                scatter(k - GAHEAD)
        for p in range(len(chunks) - GAHEAD, len(chunks)):
            scatter(p)
        for p in range(len(chunks) - NSLOT, len(chunks)):
            s_h[p].wait()
        return carry

    lax.fori_loop(0, N_SUP, super_body, 0)


def kernel(dayofweek, month, quarter, hour, session, dow_emb, month_emb,
           quarter_emb, hour_emb, session_emb, W, b):
    table = _build_table(dow_emb, month_emb, quarter_emb, hour_emb,
                         session_emb, W, b.reshape(1, D_MODEL))
    fidx = _fuse_idx(dayofweek.astype(jnp.int32), month.astype(jnp.int32),
                     quarter.astype(jnp.int32), hour.astype(jnp.int32),
                     session.astype(jnp.int32))
    out = _sc_gather(fidx, table)
    return out.reshape(B, S, D_MODEL)
